# baseline (speedup 1.0000x reference)
import functools

import jax
import jax.numpy as jnp
from jax import lax
from jax.experimental import pallas as pl
from jax.experimental.pallas import tpu as pltpu
from jax.experimental.pallas import tpu_sc as plsc

N = 10000
F = 128
H = 128
C = 40

NC = 2
NS = 16
NW = NC * NS
K = 80
NPAD = 10240
RPT = NPAD // NS


def _sc_mesh():
    return plsc.VectorSubcoreMesh(core_axis_name="c", subcore_axis_name="s")


def _make_deg_kernel(nb):
    @functools.partial(
        pl.kernel,
        mesh=_sc_mesh(),
        out_type=jax.ShapeDtypeStruct((NC, NPAD), jnp.float32),
        scratch_types=[
            pltpu.VMEM((nb, K), jnp.int32),
            pltpu.VMEM((nb, K), jnp.float32),
            pltpu.VMEM((RPT,), jnp.float32),
            pltpu.VMEM_SHARED((NPAD,), jnp.float32),
        ],
    )
    def deg_kernel(dst_hbm, w_hbm, out_hbm, dst_v, w_v, zbuf, deg_sh):
        c = lax.axis_index("c")
        s = lax.axis_index("s")
        wid = s * NC + c
        pltpu.sync_copy(dst_hbm.at[wid], dst_v)
        pltpu.sync_copy(w_hbm.at[wid], w_v)
        for i in range(RPT // 16):
            zbuf[pl.ds(i * 16, 16)] = jnp.zeros((16,), jnp.float32)
        pltpu.sync_copy(zbuf, deg_sh.at[pl.ds(s * RPT, RPT)])
        plsc.subcore_barrier()

        def body(j, carry):
            pltpu.sync_copy(w_v.at[j], deg_sh.at[dst_v.at[j]], add=True)
            return carry

        lax.fori_loop(0, nb, body, 0)
        plsc.subcore_barrier()
        pltpu.sync_copy(deg_sh.at[pl.ds(s * RPT, RPT)],
                        out_hbm.at[c, pl.ds(s * RPT, RPT)])

    return deg_kernel


NRING = 4
IRING = 8


def _make_agg_kernel(nb):
    @functools.partial(
        pl.kernel,
        mesh=_sc_mesh(),
        out_type=jax.ShapeDtypeStruct((NC, NPAD, H), jnp.float32),
        scratch_types=[
            pltpu.VMEM((IRING, 2, K), jnp.int32),
            pltpu.VMEM((IRING, K), jnp.float32),
            pltpu.VMEM((NRING, K, H), jnp.float32),
            pltpu.VMEM_SHARED((NPAD, H), jnp.float32),
            pltpu.SemaphoreType.DMA((NRING,)),
            pltpu.SemaphoreType.DMA((NRING,)),
            pltpu.SemaphoreType.DMA((IRING,)),
        ],
    )
    def agg_kernel(g_hbm, e_hbm, w_hbm, out_hbm, idx, wring, rows, acc_sh,
                   gsem, ssem, isem):
        c = lax.axis_index("c")
        s = lax.axis_index("s")
        wid = s * NC + c

        def load_idx(j):
            islot = j & (IRING - 1)
            pltpu.async_copy(e_hbm.at[wid, j], idx.at[islot],
                             isem.at[islot])
            pltpu.async_copy(w_hbm.at[wid, j], wring.at[islot],
                             isem.at[islot])

        def wait_idx(j):
            islot = j & (IRING - 1)
            pltpu.make_async_copy(e_hbm.at[wid, j], idx.at[islot],
                                  isem.at[islot]).wait()
            pltpu.make_async_copy(w_hbm.at[wid, j], wring.at[islot],
                                  isem.at[islot]).wait()

        def start_gather(j):
            islot = j & (IRING - 1)
            p = j & (NRING - 1)
            pltpu.async_copy(g_hbm.at[idx.at[islot, 0]],
                             rows.at[p], gsem.at[p])

        def wait_gather(j):
            islot = j & (IRING - 1)
            p = j & (NRING - 1)
            pltpu.make_async_copy(g_hbm.at[idx.at[islot, 0]],
                                  rows.at[p], gsem.at[p]).wait()

        def start_scatter(j):
            islot = j & (IRING - 1)
            p = j & (NRING - 1)
            pltpu.async_copy(rows.at[p],
                             acc_sh.at[idx.at[islot, 1]],
                             ssem.at[p], add=True)

        def wait_scatter(j):
            islot = j & (IRING - 1)
            p = j & (NRING - 1)
            pltpu.make_async_copy(rows.at[p],
                                  acc_sh.at[idx.at[islot, 1]],
                                  ssem.at[p]).wait()

        def scale(j):
            p = j & (NRING - 1)
            islot = j & (IRING - 1)

            def sbody(eb, inner):
                base = eb * 16
                wv = wring[islot, pl.ds(base, 16)]
                for l in range(16):
                    wsc = wv[l]
                    for q in range(H // 16):
                        sl = pl.ds(q * 16, 16)
                        rows[p, base + l, sl] = rows[p, base + l, sl] * wsc
                return inner

            lax.fori_loop(0, K // 16, sbody, 0)

        def zbody(j, carry):
            for q in range(H // 16):
                rows[0, j, pl.ds(q * 16, 16)] = jnp.zeros((16,), jnp.float32)
            return carry

        lax.fori_loop(0, K, zbody, 0)
        for t in range(RPT // K):
            pltpu.sync_copy(rows.at[0], acc_sh.at[pl.ds(s * RPT + t * K, K)])
        plsc.subcore_barrier()

        for jj in range(4):
            load_idx(jj)
        for jj in range(2):
            wait_idx(jj)
            start_gather(jj)
        for jj in range(2):
            wait_gather(jj)
            scale(jj)
            start_scatter(jj)
            load_idx(jj + 4)
            wait_idx(jj + 2)
            start_gather(jj + 2)

        def body(j, carry):
            wait_gather(j)
            scale(j)
            start_scatter(j)

            @pl.when(j + 4 < nb)
            def _():
                load_idx(j + 4)

            @pl.when(j + 2 < nb)
            def _():
                wait_scatter(j - 2)
                wait_idx(j + 2)
                start_gather(j + 2)

            return carry

        lax.fori_loop(2, nb, body, 0)
        for jj in range(nb - NRING, nb):
            wait_scatter(jj)
        plsc.subcore_barrier()
        sl = pl.ds(s * RPT, RPT)
        pltpu.sync_copy(acc_sh.at[sl], out_hbm.at[c, sl])

    return agg_kernel


def _tc_grid(nblk, rows):
    return dict(grid=(nblk,)), rows


_BLK = 1000
_NBLK = N // _BLK


def _rowspec():
    return pl.BlockSpec((_BLK, H), lambda i: (i, 0))


def _fullspec():
    return pl.BlockSpec((H, H), lambda i: (0, 0))


def _biasspec():
    return pl.BlockSpec((1, H), lambda i: (0, 0))


def _degspec():
    return pl.BlockSpec((NC, _BLK, 1), lambda i: (0, i, 0))


def _partspec():
    return pl.BlockSpec((NC, _BLK, H), lambda i: (0, i, 0))


def _tc1_body(x_ref, w1_ref, b1_ref, wc1_ref, degp_ref,
              h_ref, hw1_ref, g1_ref):
    deg = degp_ref[0] + degp_ref[1] + 1.0
    dinv = lax.rsqrt(deg)
    h = jnp.maximum(
        jnp.dot(x_ref[...], w1_ref[...], preferred_element_type=jnp.float32)
        + b1_ref[...], 0.0)
    hw1 = jnp.dot(h, wc1_ref[...], preferred_element_type=jnp.float32)
    h_ref[...] = h
    hw1_ref[...] = hw1
    g1_ref[...] = hw1 * dinv


def _tc2_body(p_ref, hw1_ref, bc1_ref, wc2_ref, degp_ref,
              z1_ref, hw2_ref, g2_ref):
    deg = degp_ref[0] + degp_ref[1] + 1.0
    dinv = lax.rsqrt(deg)
    deginv = dinv * dinv
    acc = p_ref[0] + p_ref[1]
    hw1 = hw1_ref[...]
    z1 = jnp.maximum(acc * dinv + hw1 * deginv + bc1_ref[...], 0.0)
    hw2 = jnp.dot(z1, wc2_ref[...], preferred_element_type=jnp.float32)
    z1_ref[...] = z1
    hw2_ref[...] = hw2
    g2_ref[...] = hw2 * dinv


def _tc3_body(p_ref, hw2_ref, bc2_ref, h_ref, z1_ref, w2_ref, b2_ref,
              degp_ref, out_ref):
    deg = degp_ref[0] + degp_ref[1] + 1.0
    dinv = lax.rsqrt(deg)
    deginv = dinv * dinv
    acc = p_ref[0] + p_ref[1]
    h2 = acc * dinv + hw2_ref[...] * deginv + bc2_ref[...]
    xj = jnp.maximum(jnp.maximum(h_ref[...], z1_ref[...]), h2)
    logits = jnp.dot(xj, w2_ref[...], preferred_element_type=jnp.float32) \
        + b2_ref[...]
    m = jnp.max(logits, axis=-1, keepdims=True)
    e = jnp.exp(logits - m)
    lse = jnp.log(jnp.sum(e, axis=-1, keepdims=True))
    out_ref[...] = logits - m - lse


def _row_out(n=1):
    sh = jax.ShapeDtypeStruct((N, H), jnp.float32)
    return [sh] * n


def kernel(x, edge_index, edge_weight, W1, b1, Wc1, bc1, Wc2, bc2, W2, b2):
    e = edge_index.shape[1]
    nb = -(-e // (NW * K))
    epad = NW * nb * K
    pad = epad - e

    src = edge_index[0].astype(jnp.int32)
    dst = edge_index[1].astype(jnp.int32)
    w = edge_weight.astype(jnp.float32)
    if pad:
        fill = (jnp.arange(pad, dtype=jnp.int32) * 37) % N
        src = jnp.concatenate([src, fill])
        dst = jnp.concatenate([dst, fill])
        w = jnp.concatenate([w, jnp.zeros((pad,), jnp.float32)])
    srcs = src.reshape(NW, nb, K)
    dsts = dst.reshape(NW, nb, K)
    ws = w.reshape(NW, nb, K)
    epk = jnp.stack([srcs, dsts], axis=2)

    degp = _make_deg_kernel(nb)(dsts, ws)
    degp = degp.reshape(NC, NPAD, 1)

    b1r = b1.reshape(1, H)
    bc1r = bc1.reshape(1, H)
    bc2r = bc2.reshape(1, H)
    w2p = jnp.zeros((H, H), jnp.float32).at[:, :C].set(W2)
    b2p = jnp.full((1, H), -1e30, jnp.float32).at[0, :C].set(b2)

    h, hw1, g1 = pl.pallas_call(
        _tc1_body,
        grid=(_NBLK,),
        in_specs=[_rowspec(), _fullspec(), _biasspec(), _fullspec(),
                  _degspec()],
        out_specs=[_rowspec(), _rowspec(), _rowspec()],
        out_shape=_row_out(3),
    )(x, W1, b1r, Wc1, degp)

    p1 = _make_agg_kernel(nb)(g1, epk, ws)

    z1, hw2, g2 = pl.pallas_call(
        _tc2_body,
        grid=(_NBLK,),
        in_specs=[_partspec(), _rowspec(), _biasspec(), _fullspec(),
                  _degspec()],
        out_specs=[_rowspec(), _rowspec(), _rowspec()],
        out_shape=_row_out(3),
    )(p1, hw1, bc1r, Wc2, degp)

    p2 = _make_agg_kernel(nb)(g2, epk, ws)

    out = pl.pallas_call(
        _tc3_body,
        grid=(_NBLK,),
        in_specs=[_partspec(), _rowspec(), _biasspec(), _rowspec(),
                  _rowspec(), _fullspec(), _biasspec(), _degspec()],
        out_specs=_rowspec(),
        out_shape=jax.ShapeDtypeStruct((N, H), jnp.float32),
    )(p2, hw2, bc2r, h, z1, w2p, b2p, degp)

    return out[:, :C]


# speedup vs baseline: 1.8859x; 1.8859x over previous
import functools

import jax
import jax.numpy as jnp
from jax import lax
from jax.experimental import pallas as pl
from jax.experimental.pallas import tpu as pltpu
from jax.experimental.pallas import tpu_sc as plsc

N = 10000
F = 128
H = 128
C = 40

NC = 2
NS = 16
NW = NC * NS
K = 128
NPAD = 10240
RPT = NPAD // NS


def _sc_mesh():
    return plsc.VectorSubcoreMesh(core_axis_name="c", subcore_axis_name="s")


def _make_deg_kernel(nb):
    @functools.partial(
        pl.kernel,
        mesh=_sc_mesh(),
        out_type=jax.ShapeDtypeStruct((NC, NPAD), jnp.float32),
        scratch_types=[
            pltpu.VMEM((nb, K), jnp.int32),
            pltpu.VMEM((nb, K), jnp.float32),
            pltpu.VMEM((RPT,), jnp.float32),
            pltpu.VMEM_SHARED((NPAD,), jnp.float32),
        ],
    )
    def deg_kernel(dst_hbm, w_hbm, out_hbm, dst_v, w_v, zbuf, deg_sh):
        c = lax.axis_index("c")
        s = lax.axis_index("s")
        wid = s * NC + c
        pltpu.sync_copy(dst_hbm.at[wid], dst_v)
        pltpu.sync_copy(w_hbm.at[wid], w_v)
        for i in range(RPT // 16):
            zbuf[pl.ds(i * 16, 16)] = jnp.zeros((16,), jnp.float32)
        pltpu.sync_copy(zbuf, deg_sh.at[pl.ds(s * RPT, RPT)])
        plsc.subcore_barrier()

        def body(j, carry):
            pltpu.sync_copy(w_v.at[j], deg_sh.at[dst_v.at[j]], add=True)
            return carry

        lax.fori_loop(0, nb, body, 0)
        plsc.subcore_barrier()
        pltpu.sync_copy(deg_sh.at[pl.ds(s * RPT, RPT)],
                        out_hbm.at[c, pl.ds(s * RPT, RPT)])

    return deg_kernel


SB = 8


def _make_agg_kernel(nb):
    nsb = nb // SB
    nbody = nb // 2

    @functools.partial(
        pl.kernel,
        mesh=_sc_mesh(),
        out_type=jax.ShapeDtypeStruct((NC, NPAD, H), jnp.float32),
        scratch_types=[
            pltpu.VMEM((2, SB, 2, K), jnp.int32),
            pltpu.VMEM((2, SB, K), jnp.float32),
            pltpu.VMEM((K, H), jnp.float32),
            pltpu.VMEM((K, H), jnp.float32),
            pltpu.VMEM_SHARED((NPAD, H), jnp.float32),
            pltpu.SemaphoreType.DMA,
            pltpu.SemaphoreType.DMA,
            pltpu.SemaphoreType.DMA,
            pltpu.SemaphoreType.DMA,
            pltpu.SemaphoreType.DMA((2,)),
        ],
    )
    def agg_kernel(g_hbm, e_hbm, w_hbm, out_hbm, sbi, sbw, ga, gb,
                   acc_sh, gsa, gsb, ssa, ssb, isem):
        c = lax.axis_index("c")
        s = lax.axis_index("s")
        wid = s * NC + c

        def load_sb(sb, slot):
            pltpu.async_copy(e_hbm.at[wid, pl.ds(sb * SB, SB)],
                             sbi.at[slot], isem.at[slot])
            pltpu.async_copy(w_hbm.at[wid, pl.ds(sb * SB, SB)],
                             sbw.at[slot], isem.at[slot])

        def wait_sb(sb, slot):
            pltpu.make_async_copy(e_hbm.at[wid, pl.ds(sb * SB, SB)],
                                  sbi.at[slot], isem.at[slot]).wait()
            pltpu.make_async_copy(w_hbm.at[wid, pl.ds(sb * SB, SB)],
                                  sbw.at[slot], isem.at[slot]).wait()

        def start_gather(slot, t, buf, sem):
            pltpu.async_copy(g_hbm.at[sbi.at[slot, t, 0]], buf, sem)

        def wait_gather(buf, sem):
            pltpu.make_async_copy(g_hbm.at[sbi.at[0, 0, 0]], buf, sem).wait()

        def start_scatter(slot, t, buf, sem):
            pltpu.async_copy(buf, acc_sh.at[sbi.at[slot, t, 1]], sem,
                             add=True)

        def wait_scatter(buf, sem):
            pltpu.make_async_copy(buf, acc_sh.at[sbi.at[0, 0, 1]], sem).wait()

        def scale(slot, t, src, dst):
            def sbody(eb, inner):
                base = eb * 16
                wvec = sbw[slot, t, pl.ds(base, 16)]
                for l in range(16):
                    wsc = wvec[l]
                    for q in range(H // 16):
                        sl = pl.ds(q * 16, 16)
                        dst[base + l, sl] = src[base + l, sl] * wsc
                return inner

            lax.fori_loop(0, K // 16, sbody, 0)

        def zbody(j, carry):
            for q in range(H // 16):
                ga[j, pl.ds(q * 16, 16)] = jnp.zeros((16,), jnp.float32)
            return carry

        lax.fori_loop(0, K, zbody, 0)
        zbase = s * RPT
        for t in range(RPT // K):
            pltpu.sync_copy(ga, acc_sh.at[pl.ds(zbase + t * K, K)])
        plsc.subcore_barrier()

        load_sb(0, 0)
        wait_sb(0, 0)

        def body(i, carry):
            sb = lax.shift_right_logical(i, 2)
            sbpos = lax.bitwise_and(i, 3)
            slot = lax.bitwise_and(sb, 1)
            ta = sbpos * 2
            tb = ta + 1
            i1 = i + 1
            sbpos1 = lax.bitwise_and(i1, 3)
            slot1 = lax.bitwise_and(lax.shift_right_logical(i1, 2), 1)
            t1a = sbpos1 * 2
            t1b = t1a + 1

            @pl.when((sbpos == 0) & (sb + 1 < nsb))
            def _():
                load_sb(sb + 1, 1 - slot)

            @pl.when((sbpos == 3) & (sb + 1 < nsb))
            def _():
                wait_sb(sb + 1, 1 - slot)

            start_gather(slot, ta, ga, gsa)
            wait_gather(ga, gsa)
            scale(slot, ta, ga, ga)

            @pl.when(i > 0)
            def _():
                wait_scatter(ga, ssa)

            start_scatter(slot, ta, ga, ssa)

            start_gather(slot, tb, gb, gsb)
            wait_gather(gb, gsb)
            scale(slot, tb, gb, gb)

            @pl.when(i > 0)
            def _():
                wait_scatter(gb, ssb)

            start_scatter(slot, tb, gb, ssb)
            return carry

        lax.fori_loop(0, nbody, body, 0)
        wait_scatter(ga, ssa)
        wait_scatter(gb, ssb)
        plsc.subcore_barrier()
        sl = pl.ds(s * RPT, RPT)
        pltpu.sync_copy(acc_sh.at[sl], out_hbm.at[c, sl])

    return agg_kernel

def _tc_grid(nblk, rows):
    return dict(grid=(nblk,)), rows


_BLK = 1000
_NBLK = N // _BLK


def _rowspec():
    return pl.BlockSpec((_BLK, H), lambda i: (i, 0))


def _fullspec():
    return pl.BlockSpec((H, H), lambda i: (0, 0))


def _biasspec():
    return pl.BlockSpec((1, H), lambda i: (0, 0))


def _degspec():
    return pl.BlockSpec((NC, _BLK, 1), lambda i: (0, i, 0))


def _partspec():
    return pl.BlockSpec((NC, _BLK, H), lambda i: (0, i, 0))


def _tc1_body(x_ref, w1_ref, b1_ref, wc1_ref, degp_ref,
              h_ref, hw1_ref, g1_ref):
    deg = degp_ref[0] + degp_ref[1] + 1.0
    dinv = lax.rsqrt(deg)
    h = jnp.maximum(
        jnp.dot(x_ref[...], w1_ref[...], preferred_element_type=jnp.float32)
        + b1_ref[...], 0.0)
    hw1 = jnp.dot(h, wc1_ref[...], preferred_element_type=jnp.float32)
    h_ref[...] = h
    hw1_ref[...] = hw1
    g1_ref[...] = hw1 * dinv


def _tc2_body(p_ref, hw1_ref, bc1_ref, wc2_ref, degp_ref,
              z1_ref, hw2_ref, g2_ref):
    deg = degp_ref[0] + degp_ref[1] + 1.0
    dinv = lax.rsqrt(deg)
    deginv = dinv * dinv
    acc = p_ref[0] + p_ref[1]
    hw1 = hw1_ref[...]
    z1 = jnp.maximum(acc * dinv + hw1 * deginv + bc1_ref[...], 0.0)
    hw2 = jnp.dot(z1, wc2_ref[...], preferred_element_type=jnp.float32)
    z1_ref[...] = z1
    hw2_ref[...] = hw2
    g2_ref[...] = hw2 * dinv


def _tc3_body(p_ref, hw2_ref, bc2_ref, h_ref, z1_ref, w2_ref, b2_ref,
              degp_ref, out_ref):
    deg = degp_ref[0] + degp_ref[1] + 1.0
    dinv = lax.rsqrt(deg)
    deginv = dinv * dinv
    acc = p_ref[0] + p_ref[1]
    h2 = acc * dinv + hw2_ref[...] * deginv + bc2_ref[...]
    xj = jnp.maximum(jnp.maximum(h_ref[...], z1_ref[...]), h2)
    logits = jnp.dot(xj, w2_ref[...], preferred_element_type=jnp.float32) \
        + b2_ref[...]
    m = jnp.max(logits, axis=-1, keepdims=True)
    e = jnp.exp(logits - m)
    lse = jnp.log(jnp.sum(e, axis=-1, keepdims=True))
    out_ref[...] = logits - m - lse


def _row_out(n=1):
    sh = jax.ShapeDtypeStruct((N, H), jnp.float32)
    return [sh] * n


def kernel(x, edge_index, edge_weight, W1, b1, Wc1, bc1, Wc2, bc2, W2, b2):
    e = edge_index.shape[1]
    nb = -(-e // (NW * K))
    nb = -(-nb // SB) * SB
    epad = NW * nb * K
    pad = epad - e

    src = edge_index[0].astype(jnp.int32)
    dst = edge_index[1].astype(jnp.int32)
    w = edge_weight.astype(jnp.float32)
    if pad:
        fill = (jnp.arange(pad, dtype=jnp.int32) * 37) % N
        src = jnp.concatenate([src, fill])
        dst = jnp.concatenate([dst, fill])
        w = jnp.concatenate([w, jnp.zeros((pad,), jnp.float32)])
    srcs = src.reshape(NW, nb, K)
    dsts = dst.reshape(NW, nb, K)
    ws = w.reshape(NW, nb, K)
    epk = jnp.stack([srcs, dsts], axis=2)

    degp = _make_deg_kernel(nb)(dsts, ws)
    degp = degp.reshape(NC, NPAD, 1)

    b1r = b1.reshape(1, H)
    bc1r = bc1.reshape(1, H)
    bc2r = bc2.reshape(1, H)
    w2p = jnp.zeros((H, H), jnp.float32).at[:, :C].set(W2)
    b2p = jnp.full((1, H), -1e30, jnp.float32).at[0, :C].set(b2)

    h, hw1, g1 = pl.pallas_call(
        _tc1_body,
        grid=(_NBLK,),
        in_specs=[_rowspec(), _fullspec(), _biasspec(), _fullspec(),
                  _degspec()],
        out_specs=[_rowspec(), _rowspec(), _rowspec()],
        out_shape=_row_out(3),
    )(x, W1, b1r, Wc1, degp)

    p1 = _make_agg_kernel(nb)(g1, epk, ws)

    z1, hw2, g2 = pl.pallas_call(
        _tc2_body,
        grid=(_NBLK,),
        in_specs=[_partspec(), _rowspec(), _biasspec(), _fullspec(),
                  _degspec()],
        out_specs=[_rowspec(), _rowspec(), _rowspec()],
        out_shape=_row_out(3),
    )(p1, hw1, bc1r, Wc2, degp)

    p2 = _make_agg_kernel(nb)(g2, epk, ws)

    out = pl.pallas_call(
        _tc3_body,
        grid=(_NBLK,),
        in_specs=[_partspec(), _rowspec(), _biasspec(), _rowspec(),
                  _rowspec(), _fullspec(), _biasspec(), _degspec()],
        out_specs=_rowspec(),
        out_shape=jax.ShapeDtypeStruct((N, H), jnp.float32),
    )(p2, hw2, bc2r, h, z1, w2p, b2p, degp)

    return out[:, :C]
